# Initial kernel scaffold; baseline (speedup 1.0000x reference)
import functools

import jax
import jax.numpy as jnp
from jax import lax
from jax.experimental import pallas as pl
from jax.experimental.pallas import tpu as pltpu
from jax.experimental.pallas import tpu_sc as plsc

BATCH = 4096
SEQ = 50
VOCAB = 1000
D = 1000
N = BATCH * SEQ

NUM_WORKERS = 32
PER_WORKER = N // NUM_WORKERS
CHUNK = 128
NUM_CHUNKS = PER_WORKER // CHUNK

_MESH = plsc.VectorSubcoreMesh(core_axis_name="c", subcore_axis_name="s")


@functools.partial(
    pl.kernel,
    out_type=jax.ShapeDtypeStruct((N, D), jnp.float32),
    mesh=_MESH,
    scratch_types=[
        pltpu.VMEM((CHUNK,), jnp.int32),
        pltpu.VMEM((CHUNK, D), jnp.float32),
        pltpu.SemaphoreType.DMA,
    ],
)
def _gather_rows(x_hbm, table_hbm, out_hbm, idx_v, rows_v, sem):
    wid = lax.axis_index("s") * 2 + lax.axis_index("c")
    base = wid * PER_WORKER

    def chunk_body(g, carry):
        off = pl.multiple_of(base + g * CHUNK, CHUNK)
        pltpu.sync_copy(x_hbm.at[pl.ds(off, CHUNK)], idx_v)
        pltpu.async_copy(table_hbm.at[idx_v], rows_v, sem).wait()
        pltpu.sync_copy(rows_v, out_hbm.at[pl.ds(off, CHUNK)])
        return carry

    lax.fori_loop(0, NUM_CHUNKS, chunk_body, 0)


def kernel(x, table):
    flat = _gather_rows(x.reshape(N), table)
    return flat.reshape(BATCH, SEQ, D)


# speedup vs baseline: 1.0292x; 1.0292x over previous
import functools

import jax
import jax.numpy as jnp
from jax import lax
from jax.experimental import pallas as pl
from jax.experimental.pallas import tpu as pltpu
from jax.experimental.pallas import tpu_sc as plsc

BATCH = 4096
SEQ = 50
VOCAB = 1000
D = 1000
N = BATCH * SEQ

NUM_WORKERS = 32
PER_WORKER = N // NUM_WORKERS
CHUNK = 128
NUM_CHUNKS = PER_WORKER // CHUNK

_MESH = plsc.VectorSubcoreMesh(core_axis_name="c", subcore_axis_name="s")


@functools.partial(
    pl.kernel,
    out_type=jax.ShapeDtypeStruct((N, D), jnp.float32),
    mesh=_MESH,
    scratch_types=[
        pltpu.VMEM((CHUNK,), jnp.int32),
        pltpu.VMEM((CHUNK, D), jnp.float32),
        pltpu.SemaphoreType.DMA,
    ],
    compiler_params=pltpu.CompilerParams(use_tc_tiling_on_sc=False),
)
def _gather_rows(x_hbm, table_hbm, out_hbm, idx_v, rows_v, sem):
    wid = lax.axis_index("s") * 2 + lax.axis_index("c")
    base = wid * PER_WORKER

    def chunk_body(g, carry):
        off = pl.multiple_of(base + g * CHUNK, CHUNK)
        pltpu.sync_copy(x_hbm.at[pl.ds(off, CHUNK)], idx_v)
        pltpu.async_copy(table_hbm.at[idx_v], rows_v, sem).wait()
        pltpu.sync_copy(rows_v, out_hbm.at[pl.ds(off, CHUNK)])
        return carry

    lax.fori_loop(0, NUM_CHUNKS, chunk_body, 0)


def kernel(x, table):
    flat = _gather_rows(x.reshape(N), table)
    return flat.reshape(BATCH, SEQ, D)


# speedup vs baseline: 1.0363x; 1.0069x over previous
import functools

import jax
import jax.numpy as jnp
from jax import lax
from jax.experimental import pallas as pl
from jax.experimental.pallas import tpu as pltpu
from jax.experimental.pallas import tpu_sc as plsc

BATCH = 4096
SEQ = 50
VOCAB = 1000
D = 1000
N = BATCH * SEQ

NUM_WORKERS = 32
PER_WORKER = N // NUM_WORKERS
CHUNK = 40
NBUF = 2
NUM_CHUNKS = PER_WORKER // CHUNK

_MESH = plsc.VectorSubcoreMesh(core_axis_name="c", subcore_axis_name="s")


@functools.partial(
    pl.kernel,
    out_type=jax.ShapeDtypeStruct((N, D), jnp.float32),
    mesh=_MESH,
    scratch_types=[
        pltpu.VMEM((PER_WORKER,), jnp.int32),
        pltpu.VMEM((NBUF, CHUNK, D), jnp.float32),
        pltpu.SemaphoreType.DMA((NBUF,)),
        pltpu.SemaphoreType.DMA((NBUF,)),
    ],
    compiler_params=pltpu.CompilerParams(use_tc_tiling_on_sc=False),
)
def _gather_rows(x_hbm, table_hbm, out_hbm, idx_all, rows, sem_g, sem_w):
    wid = lax.axis_index("s") * 2 + lax.axis_index("c")
    base = wid * PER_WORKER

    def idx_slice(c):
        return idx_all.at[pl.ds(pl.multiple_of(c * CHUNK, CHUNK), CHUNK)]

    def out_slice(c):
        return out_hbm.at[pl.ds(pl.multiple_of(base + c * CHUNK, CHUNK), CHUNK)]

    pltpu.sync_copy(x_hbm.at[pl.ds(base, PER_WORKER)], idx_all)

    def gather_copy(c, b):
        return pltpu.make_async_copy(table_hbm.at[idx_slice(c)], rows.at[b],
                                     sem_g.at[b])

    def write_copy(c, b):
        return pltpu.make_async_copy(rows.at[b], out_slice(c), sem_w.at[b])

    for b in range(NBUF):
        gather_copy(b, b).start()

    def outer(go, carry):
        for b in range(NBUF):
            c = go + b
            gather_copy(c, b).wait()
            write_copy(c, b).start()

            @pl.when(c + NBUF < NUM_CHUNKS)
            def _():
                write_copy(c, b).wait()
                gather_copy(c + NBUF, b).start()

        return carry

    lax.fori_loop(0, NUM_CHUNKS // NBUF, lambda i, cr: outer(i * NBUF, cr), 0)

    for b in range(NBUF):
        write_copy(NUM_CHUNKS - NBUF + b, b).wait()


def kernel(x, table):
    flat = _gather_rows(x.reshape(N), table)
    return flat.reshape(BATCH, SEQ, D)
